# Initial kernel scaffold; baseline (speedup 1.0000x reference)
import functools

import jax
import jax.numpy as jnp
from jax import lax
from jax.experimental import pallas as pl
from jax.experimental.pallas import tpu as pltpu
from jax.experimental.pallas import tpu_sc as plsc

N = 10000
E = 320000
D = 128
NPAD = 10240
K = 80
NCH = E // K
TILES = 16
CH_T = NCH // TILES
ROWS_T = N // TILES
DROWS_T = NPAD // TILES

_mesh = plsc.VectorSubcoreMesh(core_axis_name="c", subcore_axis_name="s")


def _rsqrt_newton(x):
    i = plsc.bitcast(x, jnp.int32)
    i = 0x5F3759DF - lax.shift_right_arithmetic(i, 1)
    y = plsc.bitcast(i, jnp.float32)
    for _ in range(3):
        y = y * (1.5 - 0.5 * x * y * y)
    return y


@functools.partial(
    pl.kernel,
    out_type=jax.ShapeDtypeStruct((2, NPAD, D), jnp.float32),
    mesh=_mesh,
    scratch_types=[
        pltpu.VMEM((2000,), jnp.int32),
        pltpu.VMEM((NPAD,), jnp.float32),
        pltpu.VMEM((DROWS_T,), jnp.float32),
        pltpu.VMEM((DROWS_T,), jnp.float32),
        pltpu.VMEM((DROWS_T,), jnp.float32),
        pltpu.VMEM((DROWS_T, D), jnp.float32),
        pltpu.VMEM_SHARED((TILES, NPAD), jnp.float32),
    ],
)
def _sc_degree(dst_hbm, dinv_hbm, dbuf, hist, sumb, tmpb, dvb, bbuf, shist):
    c = lax.axis_index("c")
    s = lax.axis_index("s")
    zeros16 = jnp.zeros((16,), jnp.float32)
    ones16 = jnp.ones((16,), jnp.float32)

    def zero_body(i, _):
        hist[pl.ds(i * 16, 16)] = zeros16
        return 0
    lax.fori_loop(0, NPAD // 16, zero_body, 0)

    base = s * (E // TILES)

    def chunk_body(j, _):
        pltpu.sync_copy(dst_hbm.at[c, pl.ds(base + j * 2000, 2000)], dbuf)

        def vec_body(t, _):
            idx = dbuf[pl.ds(t * 16, 16)]
            plsc.addupdate_scatter(hist, [idx], ones16)
            return 0
        lax.fori_loop(0, 125, vec_body, 0)
        return 0
    lax.fori_loop(0, (E // TILES) // 2000, chunk_body, 0)

    pltpu.sync_copy(hist, shist.at[s])
    plsc.subcore_barrier()

    r0 = s * DROWS_T
    pltpu.sync_copy(shist.at[0, pl.ds(r0, DROWS_T)], sumb)
    for t in range(1, TILES):
        pltpu.sync_copy(shist.at[t, pl.ds(r0, DROWS_T)], tmpb)

        def add_body(i, _):
            sl = pl.ds(i * 16, 16)
            sumb[sl] = sumb[sl] + tmpb[sl]
            return 0
        lax.fori_loop(0, DROWS_T // 16, add_body, 0)

    def dinv_body(i, _):
        sl = pl.ds(i * 16, 16)
        deg = sumb[sl] + 1.0
        dvb[sl] = _rsqrt_newton(deg)
        return 0
    lax.fori_loop(0, DROWS_T // 16, dinv_body, 0)

    def bcast_body(r, _):
        d = dvb[r]
        row = jnp.broadcast_to(d, (16,))
        for j in range(D // 16):
            bbuf[r, pl.ds(j * 16, 16)] = row
        return 0
    lax.fori_loop(0, DROWS_T, bcast_body, 0)

    pltpu.sync_copy(bbuf, dinv_hbm.at[c, pl.ds(r0, DROWS_T)])


@functools.partial(
    pl.kernel,
    out_type=jax.ShapeDtypeStruct((2, NPAD, D), jnp.float32),
    mesh=_mesh,
    scratch_types=[
        pltpu.VMEM((K,), jnp.int32),
        pltpu.VMEM((K,), jnp.int32),
        pltpu.VMEM((K, D), jnp.float32),
        pltpu.VMEM((125, D), jnp.float32),
        pltpu.VMEM_SHARED((N, D), jnp.float32),
        pltpu.SemaphoreType.DMA,
    ],
)
def _sc_edge_acc(src_hbm, dst_hbm, g_hbm, out_hbm, sbuf, dbuf, rows, stg, acc, sem):
    c = lax.axis_index("c")
    s = lax.axis_index("s")
    zeros16 = jnp.zeros((16,), jnp.float32)

    def zstg_body(i, _):
        for j in range(D // 16):
            stg[i, pl.ds(j * 16, 16)] = zeros16
        return 0
    lax.fori_loop(0, 125, zstg_body, 0)
    for i in range(ROWS_T // 125):
        pltpu.sync_copy(stg, acc.at[pl.ds(s * ROWS_T + i * 125, 125)])
    plsc.subcore_barrier()

    goff = c * NPAD

    def chunk_body(j, _):
        cj = s * CH_T + j
        pltpu.sync_copy(src_hbm.at[c, cj], sbuf)
        pltpu.sync_copy(dst_hbm.at[c, cj], dbuf)
        for t in range(K // 16):
            sl = pl.ds(t * 16, 16)
            sbuf[sl] = sbuf[sl] + goff
        pltpu.async_copy(g_hbm.at[sbuf], rows, sem).wait()
        pltpu.sync_copy(rows, acc.at[dbuf], add=True)
        return 0
    lax.fori_loop(0, CH_T, chunk_body, 0)

    plsc.subcore_barrier()
    for i in range(ROWS_T // 125):
        r = s * ROWS_T + i * 125
        pltpu.sync_copy(acc.at[pl.ds(r, 125)], stg)
        pltpu.sync_copy(stg, out_hbm.at[c, pl.ds(r, 125)])


def _tc_prep_body(x_ref, h_ref, w_ref, dinv_ref, g_ref):
    z = x_ref[0] + 0.8 * h_ref[0]
    hh = jnp.dot(z, w_ref[...], preferred_element_type=jnp.float32)
    g_ref[0] = hh * dinv_ref[0]


def _tc_finish_body(a_ref, g_ref, dinv_ref, b_ref, o_ref):
    y = (a_ref[0] + g_ref[0]) * dinv_ref[0] + b_ref[...]
    o_ref[0] = jnp.maximum(y, 0.01 * y)


_RB = 1024


def _tc_prep(x, h, w, dinv):
    grid = (2, NPAD // _RB)
    blk = pl.BlockSpec((1, _RB, D), lambda g, j: (g, j, 0))
    return pl.pallas_call(
        _tc_prep_body,
        grid=grid,
        in_specs=[blk, blk, pl.BlockSpec((D, D), lambda g, j: (0, 0)), blk],
        out_specs=blk,
        out_shape=jax.ShapeDtypeStruct((2, NPAD, D), jnp.float32),
    )(x, h, w, dinv)


def _tc_finish(a, g, dinv, b):
    grid = (2, NPAD // _RB)
    blk = pl.BlockSpec((1, _RB, D), lambda gi, j: (gi, j, 0))
    return pl.pallas_call(
        _tc_finish_body,
        grid=grid,
        in_specs=[blk, blk, blk, pl.BlockSpec((D,), lambda gi, j: (0,))],
        out_specs=blk,
        out_shape=jax.ShapeDtypeStruct((2, NPAD, D), jnp.float32),
    )(a, g, dinv, b)


@jax.jit
def kernel(edge_index_1, edge_index_2, feature_A, feature_B, hidden_A, hidden_B, W, b):
    src = jnp.stack([edge_index_1[0], edge_index_2[0]]).astype(jnp.int32)
    dst = jnp.stack([edge_index_1[1], edge_index_2[1]]).astype(jnp.int32)
    src_r = src.reshape(2, NCH, K)
    dst_r = dst.reshape(2, NCH, K)

    f = jnp.stack([feature_A, feature_B])
    h = jnp.stack([hidden_A, hidden_B])
    pad = ((0, 0), (0, NPAD - N), (0, 0))
    f = jnp.pad(f, pad)
    h = jnp.pad(h, pad)

    dinv = _sc_degree(dst)
    g = _tc_prep(f, h, W, dinv)
    acc = _sc_edge_acc(src_r, dst_r, g.reshape(2 * NPAD, D))
    out = _tc_finish(acc, g, dinv, b)
    return out[0, :N], out[1, :N], hidden_A, hidden_B


# speedup vs baseline: 17.2338x; 17.2338x over previous
import functools

import jax
import jax.numpy as jnp
from jax import lax
from jax.experimental import pallas as pl
from jax.experimental.pallas import tpu as pltpu
from jax.experimental.pallas import tpu_sc as plsc

N = 10000
E = 320000
D = 128
NPAD = 10240
K = 80
NCH = E // K
TILES = 16
CH_T = NCH // TILES
ROWS_T = N // TILES
DROWS_T = NPAD // TILES

_mesh = plsc.VectorSubcoreMesh(core_axis_name="c", subcore_axis_name="s")


def _rsqrt_newton(x):
    i = plsc.bitcast(x, jnp.int32)
    i = 0x5F3759DF - lax.shift_right_arithmetic(i, 1)
    y = plsc.bitcast(i, jnp.float32)
    for _ in range(3):
        y = y * (1.5 - 0.5 * x * y * y)
    return y


_sc_params = pltpu.CompilerParams(needs_layout_passes=False)


@functools.partial(
    pl.kernel,
    out_type=jax.ShapeDtypeStruct((2, NPAD, D), jnp.float32),
    mesh=_mesh,
    compiler_params=_sc_params,
    scratch_types=[
        pltpu.VMEM((2000,), jnp.int32),
        pltpu.VMEM((NPAD,), jnp.float32),
        pltpu.VMEM((DROWS_T,), jnp.float32),
        pltpu.VMEM((DROWS_T,), jnp.float32),
        pltpu.VMEM((DROWS_T,), jnp.float32),
        pltpu.VMEM((DROWS_T, D), jnp.float32),
        pltpu.VMEM_SHARED((TILES * NPAD,), jnp.float32),
    ],
)
def _sc_degree(dst_hbm, dinv_hbm, dbuf, hist, sumb, tmpb, dvb, bbuf, shist):
    c = lax.axis_index("c")
    s = lax.axis_index("s")
    zeros16 = jnp.zeros((16,), jnp.float32)
    ones16 = jnp.ones((16,), jnp.float32)

    def zero_body(i, _):
        hist[pl.ds(i * 16, 16)] = zeros16
        return 0
    lax.fori_loop(0, NPAD // 16, zero_body, 0)

    base = s * (E // TILES)

    def chunk_body(j, _):
        pltpu.sync_copy(dst_hbm.at[pl.ds(c * E + base + j * 2000, 2000)], dbuf)

        def vec_body(t, _):
            idx = dbuf[pl.ds(t * 16, 16)]
            plsc.addupdate_scatter(hist, [idx], ones16)
            return 0
        lax.fori_loop(0, 125, vec_body, 0)
        return 0
    lax.fori_loop(0, (E // TILES) // 2000, chunk_body, 0)

    pltpu.sync_copy(hist, shist.at[pl.ds(s * NPAD, NPAD)])
    plsc.subcore_barrier()

    r0 = s * DROWS_T
    pltpu.sync_copy(shist.at[pl.ds(r0, DROWS_T)], sumb)
    for t in range(1, TILES):
        pltpu.sync_copy(shist.at[pl.ds(t * NPAD + r0, DROWS_T)], tmpb)

        def add_body(i, _):
            sl = pl.ds(i * 16, 16)
            sumb[sl] = sumb[sl] + tmpb[sl]
            return 0
        lax.fori_loop(0, DROWS_T // 16, add_body, 0)

    def dinv_body(i, _):
        sl = pl.ds(i * 16, 16)
        deg = sumb[sl] + 1.0
        dvb[sl] = _rsqrt_newton(deg)
        return 0
    lax.fori_loop(0, DROWS_T // 16, dinv_body, 0)

    def bcast_body(i, _):
        v = dvb[pl.ds(i * 16, 16)]
        for lane in range(16):
            row = jnp.broadcast_to(v[lane], (16,))
            for j in range(D // 16):
                bbuf[i * 16 + lane, pl.ds(j * 16, 16)] = row
        return 0
    lax.fori_loop(0, DROWS_T // 16, bcast_body, 0)

    pltpu.sync_copy(bbuf, dinv_hbm.at[c, pl.ds(r0, DROWS_T)])


@functools.partial(
    pl.kernel,
    out_type=jax.ShapeDtypeStruct((2, NPAD, D), jnp.float32),
    mesh=_mesh,
    compiler_params=_sc_params,
    scratch_types=[
        pltpu.VMEM((K,), jnp.int32),
        pltpu.VMEM((K,), jnp.int32),
        pltpu.VMEM((K, D), jnp.float32),
        pltpu.VMEM((128, D), jnp.float32),
        pltpu.VMEM_SHARED((NPAD, D), jnp.float32),
        pltpu.SemaphoreType.DMA,
    ],
)
def _sc_edge_acc(src_hbm, dst_hbm, g_hbm, out_hbm, sbuf, dbuf, rows, stg, acc, sem):
    c = lax.axis_index("c")
    s = lax.axis_index("s")
    zeros16 = jnp.zeros((16,), jnp.float32)

    def zstg_body(i, _):
        for j in range(D // 16):
            stg[i, pl.ds(j * 16, 16)] = zeros16
        return 0
    lax.fori_loop(0, 128, zstg_body, 0)
    for i in range(DROWS_T // 128):
        pltpu.sync_copy(stg, acc.at[pl.ds(s * DROWS_T + i * 128, 128)])
    plsc.subcore_barrier()

    goff = c * NPAD

    def chunk_body(j, _):
        cj = s * CH_T + j
        pltpu.sync_copy(src_hbm.at[pl.ds(c * E + cj * K, K)], sbuf)
        pltpu.sync_copy(dst_hbm.at[pl.ds(c * E + cj * K, K)], dbuf)
        for t in range(K // 16):
            sl = pl.ds(t * 16, 16)
            sbuf[sl] = sbuf[sl] + goff
        pltpu.async_copy(g_hbm.at[sbuf], rows, sem).wait()
        pltpu.sync_copy(rows, acc.at[dbuf], add=True)
        return 0
    lax.fori_loop(0, CH_T, chunk_body, 0)

    plsc.subcore_barrier()
    for i in range(DROWS_T // 128):
        r = s * DROWS_T + i * 128
        pltpu.sync_copy(acc.at[pl.ds(r, 128)], stg)
        pltpu.sync_copy(stg, out_hbm.at[c, pl.ds(r, 128)])


def _tc_prep_body(x_ref, h_ref, w_ref, dinv_ref, g_ref):
    z = x_ref[0] + 0.8 * h_ref[0]
    hh = jnp.dot(z, w_ref[...], preferred_element_type=jnp.float32)
    g_ref[0] = hh * dinv_ref[0]


def _tc_finish_body(a_ref, g_ref, dinv_ref, b_ref, o_ref):
    y = (a_ref[0] + g_ref[0]) * dinv_ref[0] + b_ref[...]
    o_ref[0] = jnp.maximum(y, 0.01 * y)


_RB = 1024


def _tc_prep(x, h, w, dinv):
    grid = (2, NPAD // _RB)
    blk = pl.BlockSpec((1, _RB, D), lambda g, j: (g, j, 0))
    return pl.pallas_call(
        _tc_prep_body,
        grid=grid,
        in_specs=[blk, blk, pl.BlockSpec((D, D), lambda g, j: (0, 0)), blk],
        out_specs=blk,
        out_shape=jax.ShapeDtypeStruct((2, NPAD, D), jnp.float32),
    )(x, h, w, dinv)


def _tc_finish(a, g, dinv, b):
    grid = (2, NPAD // _RB)
    blk = pl.BlockSpec((1, _RB, D), lambda gi, j: (gi, j, 0))
    return pl.pallas_call(
        _tc_finish_body,
        grid=grid,
        in_specs=[blk, blk, blk, pl.BlockSpec((D,), lambda gi, j: (0,))],
        out_specs=blk,
        out_shape=jax.ShapeDtypeStruct((2, NPAD, D), jnp.float32),
    )(a, g, dinv, b)


@jax.jit
def kernel(edge_index_1, edge_index_2, feature_A, feature_B, hidden_A, hidden_B, W, b):
    src_f = jnp.concatenate([edge_index_1[0], edge_index_2[0]]).astype(jnp.int32)
    dst_f = jnp.concatenate([edge_index_1[1], edge_index_2[1]]).astype(jnp.int32)

    f = jnp.stack([feature_A, feature_B])
    h = jnp.stack([hidden_A, hidden_B])
    pad = ((0, 0), (0, NPAD - N), (0, 0))
    f = jnp.pad(f, pad)
    h = jnp.pad(h, pad)

    dinv = _sc_degree(dst_f)
    g = _tc_prep(f, h, W, dinv)
    acc = _sc_edge_acc(src_f, dst_f, g.reshape(2 * NPAD, D))
    out = _tc_finish(acc, g, dinv, b)
    return out[0, :N], out[1, :N], hidden_A, hidden_B


# speedup vs baseline: 31.2022x; 1.8105x over previous
import functools

import jax
import jax.numpy as jnp
from jax import lax
from jax.experimental import pallas as pl
from jax.experimental.pallas import tpu as pltpu
from jax.experimental.pallas import tpu_sc as plsc

N = 10000
E = 320000
D = 128
NPAD = 10240
K = 112
TILES = 16
EP = 320768
CH_T = (EP // K) // TILES
DROWS_T = NPAD // TILES
NB = 3
CHD = 2864

_mesh = plsc.VectorSubcoreMesh(core_axis_name="c", subcore_axis_name="s")


def _rsqrt_newton(x):
    i = plsc.bitcast(x, jnp.int32)
    i = 0x5F3759DF - lax.shift_right_arithmetic(i, 1)
    y = plsc.bitcast(i, jnp.float32)
    for _ in range(3):
        y = y * (1.5 - 0.5 * x * y * y)
    return y


_sc_params = pltpu.CompilerParams(needs_layout_passes=False)


@functools.partial(
    pl.kernel,
    out_type=jax.ShapeDtypeStruct((2, NPAD, D), jnp.float32),
    mesh=_mesh,
    compiler_params=_sc_params,
    scratch_types=[
        pltpu.VMEM((CHD,), jnp.int32),
        pltpu.VMEM((NPAD,), jnp.float32),
        pltpu.VMEM((DROWS_T,), jnp.float32),
        pltpu.VMEM((DROWS_T,), jnp.float32),
        pltpu.VMEM((DROWS_T,), jnp.float32),
        pltpu.VMEM((DROWS_T, D), jnp.float32),
        pltpu.VMEM_SHARED((TILES * NPAD,), jnp.float32),
    ],
)
def _sc_degree(dst_hbm, dinv_hbm, dbuf, hist, sumb, tmpb, dvb, bbuf, shist):
    c = lax.axis_index("c")
    s = lax.axis_index("s")
    zeros16 = jnp.zeros((16,), jnp.float32)
    ones16 = jnp.ones((16,), jnp.float32)

    def zero_body(i, _):
        hist[pl.ds(i * 16, 16)] = zeros16
        return 0
    lax.fori_loop(0, NPAD // 16, zero_body, 0)

    base = s * (EP // TILES)

    def chunk_body(j, _):
        pltpu.sync_copy(dst_hbm.at[pl.ds(c * EP + base + j * CHD, CHD)], dbuf)

        def vec_body(t, _):
            idx = dbuf[pl.ds(t * 16, 16)]
            plsc.addupdate_scatter(hist, [idx], ones16)
            return 0
        lax.fori_loop(0, CHD // 16, vec_body, 0)
        return 0
    lax.fori_loop(0, (EP // TILES) // CHD, chunk_body, 0)

    pltpu.sync_copy(hist, shist.at[pl.ds(s * NPAD, NPAD)])
    plsc.subcore_barrier()

    r0 = s * DROWS_T
    pltpu.sync_copy(shist.at[pl.ds(r0, DROWS_T)], sumb)
    for t in range(1, TILES):
        pltpu.sync_copy(shist.at[pl.ds(t * NPAD + r0, DROWS_T)], tmpb)

        def add_body(i, _):
            sl = pl.ds(i * 16, 16)
            sumb[sl] = sumb[sl] + tmpb[sl]
            return 0
        lax.fori_loop(0, DROWS_T // 16, add_body, 0)

    def dinv_body(i, _):
        sl = pl.ds(i * 16, 16)
        deg = sumb[sl] + 1.0
        dvb[sl] = _rsqrt_newton(deg)
        return 0
    lax.fori_loop(0, DROWS_T // 16, dinv_body, 0)

    def bcast_body(i, _):
        v = dvb[pl.ds(i * 16, 16)]
        for lane in range(16):
            row = jnp.broadcast_to(v[lane], (16,))
            for j in range(D // 16):
                bbuf[i * 16 + lane, pl.ds(j * 16, 16)] = row
        return 0
    lax.fori_loop(0, DROWS_T // 16, bcast_body, 0)

    pltpu.sync_copy(bbuf, dinv_hbm.at[c, pl.ds(r0, DROWS_T)])


@functools.partial(
    pl.kernel,
    out_type=jax.ShapeDtypeStruct((2, NPAD, D), jnp.float32),
    mesh=_mesh,
    compiler_params=_sc_params,
    scratch_types=[
        [pltpu.VMEM((K,), jnp.int32) for _ in range(NB)],
        [pltpu.VMEM((K,), jnp.int32) for _ in range(NB)],
        [pltpu.VMEM((K, D), jnp.float32) for _ in range(NB)],
        pltpu.VMEM_SHARED((NPAD, D), jnp.float32),
        [pltpu.SemaphoreType.DMA for _ in range(NB)],
        [pltpu.SemaphoreType.DMA for _ in range(NB)],
        [pltpu.SemaphoreType.DMA for _ in range(NB)],
    ],
)
def _sc_edge_acc(src_hbm, dst_hbm, g_hbm, out_hbm,
                 sbufs, dbufs, rows, acc, isems, gsems, ssems):
    c = lax.axis_index("c")
    s = lax.axis_index("s")
    zeros16 = jnp.zeros((16,), jnp.float32)

    stg = rows[0].at[pl.ds(0, 64)]

    def zstg_body(i, _):
        for j in range(D // 16):
            rows[0][i, pl.ds(j * 16, 16)] = zeros16
        return 0
    lax.fori_loop(0, 64, zstg_body, 0)
    for i in range(DROWS_T // 64):
        pltpu.sync_copy(stg, acc.at[pl.ds(s * DROWS_T + i * 64, 64)])
    plsc.subcore_barrier()

    ebase = c * EP + s * CH_T * K

    def idx_start(j, b):
        off = ebase + j * K
        pltpu.async_copy(src_hbm.at[pl.ds(off, K)], sbufs[b], isems[b])
        pltpu.async_copy(dst_hbm.at[pl.ds(off, K)], dbufs[b], isems[b])

    def idx_wait(j, b):
        off = ebase + j * K
        pltpu.make_async_copy(src_hbm.at[pl.ds(off, K)], sbufs[b], isems[b]).wait()
        pltpu.make_async_copy(dst_hbm.at[pl.ds(off, K)], dbufs[b], isems[b]).wait()

    def g_start(b):
        pltpu.async_copy(g_hbm.at[sbufs[b]], rows[b], gsems[b])

    def g_wait(b):
        pltpu.make_async_copy(g_hbm.at[sbufs[b]], rows[b], gsems[b]).wait()

    def s_start(b):
        pltpu.async_copy(rows[b], acc.at[dbufs[b]], ssems[b], add=True)

    def s_wait(b):
        pltpu.make_async_copy(rows[b], acc.at[dbufs[b]], ssems[b]).wait()

    idx_start(0, 0)
    idx_start(1, 1)
    idx_wait(0, 0)
    g_start(0)

    def chunk_body(j, _):
        for b in range(NB):
            @pl.when(j % NB == b)
            def _():
                bg = (b + 1) % NB
                bi = (b + 2) % NB
                g_wait(b)
                s_start(b)

                @pl.when(j + 1 < CH_T)
                def _():
                    idx_wait(j + 1, bg)
                    g_start(bg)

                @pl.when(j + 2 < CH_T)
                def _():
                    @pl.when(j + 2 >= NB)
                    def _():
                        s_wait(bi)
                    idx_start(j + 2, bi)
        return 0
    lax.fori_loop(0, CH_T, chunk_body, 0)
    for b in range(NB):
        s_wait(b)

    plsc.subcore_barrier()
    for i in range(DROWS_T // 64):
        r = s * DROWS_T + i * 64
        pltpu.sync_copy(acc.at[pl.ds(r, 64)], stg)
        pltpu.sync_copy(stg, out_hbm.at[c, pl.ds(r, 64)])


def _tc_prep_body(x_ref, h_ref, w_ref, dinv_ref, g_ref):
    z = x_ref[0] + 0.8 * h_ref[0]
    hh = jnp.dot(z, w_ref[...], preferred_element_type=jnp.float32)
    g_ref[0] = hh * dinv_ref[0]


def _tc_finish_body(a_ref, g_ref, dinv_ref, b_ref, o_ref):
    y = (a_ref[0] + g_ref[0]) * dinv_ref[0] + b_ref[...]
    o_ref[0] = jnp.maximum(y, 0.01 * y)


_RB = 1024


def _tc_prep(x, h, w, dinv):
    grid = (2, NPAD // _RB)
    blk = pl.BlockSpec((1, _RB, D), lambda g, j: (g, j, 0))
    return pl.pallas_call(
        _tc_prep_body,
        grid=grid,
        in_specs=[blk, blk, pl.BlockSpec((D, D), lambda g, j: (0, 0)), blk],
        out_specs=blk,
        out_shape=jax.ShapeDtypeStruct((2, NPAD, D), jnp.float32),
    )(x, h, w, dinv)


def _tc_finish(a, g, dinv, b):
    grid = (2, NPAD // _RB)
    blk = pl.BlockSpec((1, _RB, D), lambda gi, j: (gi, j, 0))
    return pl.pallas_call(
        _tc_finish_body,
        grid=grid,
        in_specs=[blk, blk, blk, pl.BlockSpec((D,), lambda gi, j: (0,))],
        out_specs=blk,
        out_shape=jax.ShapeDtypeStruct((2, NPAD, D), jnp.float32),
    )(a, g, dinv, b)


@jax.jit
def kernel(edge_index_1, edge_index_2, feature_A, feature_B, hidden_A, hidden_B, W, b):
    spad = jnp.full((EP - E,), N, dtype=jnp.int32)
    dpad = jnp.full((EP - E,), NPAD - 2, dtype=jnp.int32)
    src_f = jnp.concatenate([
        edge_index_1[0].astype(jnp.int32), spad,
        edge_index_2[0].astype(jnp.int32) + NPAD, spad + NPAD,
    ])
    dst_f = jnp.concatenate([
        edge_index_1[1].astype(jnp.int32), dpad,
        edge_index_2[1].astype(jnp.int32), dpad,
    ])

    f = jnp.stack([feature_A, feature_B])
    h = jnp.stack([hidden_A, hidden_B])
    pad = ((0, 0), (0, NPAD - N), (0, 0))
    f = jnp.pad(f, pad)
    h = jnp.pad(h, pad)

    dinv = _sc_degree(dst_f)
    g = _tc_prep(f, h, W, dinv)
    acc = _sc_edge_acc(src_f, dst_f, g.reshape(2 * NPAD, D))
    out = _tc_finish(acc, g, dinv, b)
    return out[0, :N], out[1, :N], hidden_A, hidden_B
